# baseline (speedup 1.0000x reference)
import jax
import jax.numpy as jnp
from jax.experimental import pallas as pl
from jax.experimental.pallas import tpu as pltpu

BM = 512
NB = (4 * 2048 * 2048) // (2048 * BM)


def _cast_kernel(w_ref, wbf_ref):
    wbf_ref[:] = w_ref[:].astype(jnp.bfloat16)


def _router_kernel(x_ref, w_ref, b_ref, gating_ref, idx_ref, l0, l1):
    j = pl.program_id(0)
    even = (j % 2) == 0

    def do_dot(l_ref):
        l_ref[:] = jnp.dot(x_ref[:].astype(jnp.bfloat16), w_ref[:],
                           preferred_element_type=jnp.float32)

    def do_epilogue(l_ref):
        logits = l_ref[:] + b_ref[:]
        row_max = jnp.max(logits, axis=-1, keepdims=True)
        e = jnp.exp(logits - row_max)
        denom = jnp.sum(e, axis=-1, keepdims=True)
        gating_ref[:] = e / denom
        iota = jax.lax.broadcasted_iota(jnp.int32, logits.shape, 1)
        cand = jnp.where(logits == row_max, iota, jnp.int32(2**30))
        idx_ref[:] = jnp.min(cand, axis=-1, keepdims=True)

    pl.when((j < NB) & even)(lambda: do_dot(l0))
    pl.when((j < NB) & jnp.logical_not(even))(lambda: do_dot(l1))
    pl.when((j >= 1) & jnp.logical_not(even))(lambda: do_epilogue(l0))
    pl.when((j >= 1) & even)(lambda: do_epilogue(l1))


def kernel(x, gate_W, gate_b):
    B, S, D = x.shape
    M = B * S
    x2 = x.reshape(M, D)
    b2 = gate_b.reshape(1, D)

    w_bf16 = pl.pallas_call(
        _cast_kernel,
        grid=(8,),
        in_specs=[pl.BlockSpec((D // 8, D), lambda i: (i, 0))],
        out_specs=pl.BlockSpec((D // 8, D), lambda i: (i, 0)),
        out_shape=jax.ShapeDtypeStruct((D, D), jnp.bfloat16),
    )(gate_W)

    gating, idx = pl.pallas_call(
        _router_kernel,
        grid=(NB + 1,),
        in_specs=[
            pl.BlockSpec((BM, D), lambda j: (jnp.minimum(j, NB - 1), 0)),
            pl.BlockSpec((D, D), lambda j: (0, 0)),
            pl.BlockSpec((1, D), lambda j: (0, 0)),
        ],
        out_specs=[
            pl.BlockSpec((BM, D), lambda j: (jnp.maximum(j - 1, 0), 0)),
            pl.BlockSpec((BM, 1), lambda j: (jnp.maximum(j - 1, 0), 0)),
        ],
        out_shape=[
            jax.ShapeDtypeStruct((M, D), jnp.float32),
            jax.ShapeDtypeStruct((M, 1), jnp.int32),
        ],
        scratch_shapes=[
            pltpu.VMEM((BM, D), jnp.float32),
            pltpu.VMEM((BM, D), jnp.float32),
        ],
        compiler_params=pltpu.CompilerParams(
            dimension_semantics=("arbitrary",),
        ),
    )(x2, w_bf16, b2)
    return gating.reshape(B, S, D), idx.reshape(B, S)
